# Initial kernel scaffold; baseline (speedup 1.0000x reference)
import jax
import jax.numpy as jnp
from jax.experimental import pallas as pl


def kernel(instructions, entity_emb, fact_relations, topic_label, entity_mask, batch_ids, head2edge, tail2edge, ln_g, ln_b, W_rel, b_rel, W_ih_0, W_hh_0, b_hh_0, W_ih_1, W_hh_1, b_hh_1, W_score, b_score, W_ffn, b_ffn):
    raise NotImplementedError("write your pallas kernel here")


# speedup vs baseline: 1.1231x; 1.1231x over previous
import functools

import jax
import jax.numpy as jnp
from jax import lax
from jax.experimental import pallas as pl
from jax.experimental.pallas import tpu as pltpu


D = 128


def _ln(x, g, b, eps=1e-5):
    mu = jnp.mean(x, axis=-1, keepdims=True)
    var = jnp.mean((x - mu) ** 2, axis=-1, keepdims=True)
    return (x - mu) * jax.lax.rsqrt(var + eps) * g + b


def _ln_body(x_ref, g_ref, b_ref, o_ref):
    o_ref[...] = _ln(x_ref[...], g_ref[...], b_ref[...])


def _ln_rows(x, g, b, block=1000):
    n = x.shape[0]
    grid = n // block
    return pl.pallas_call(
        _ln_body,
        grid=(grid,),
        in_specs=[
            pl.BlockSpec((block, D), lambda i: (i, 0)),
            pl.BlockSpec((1, D), lambda i: (0, 0)),
            pl.BlockSpec((1, D), lambda i: (0, 0)),
        ],
        out_specs=pl.BlockSpec((block, D), lambda i: (i, 0)),
        out_shape=jax.ShapeDtypeStruct((n, D), jnp.float32),
    )(x, g.reshape(1, D), b.reshape(1, D))


def _kge_body(u_ref, rel_ref, w_ref, b_ref, acc_ref):
    i = pl.program_id(0)

    @pl.when(i == 0)
    def _init():
        acc_ref[...] = jnp.zeros_like(acc_ref)

    u = u_ref[...]
    rel = rel_ref[...]
    v = jax.lax.dot_general(u, w_ref[...], (((1,), (0,)), ((), ())),
                            preferred_element_type=jnp.float32)
    s = jnp.sum(v * rel, axis=1) + jnp.sum(u * b_ref[...], axis=1)
    nll = -jnp.log(jax.nn.sigmoid(s) + 1e-20)
    acc_ref[...] += jnp.sum(nll).reshape(1, 1)


def _kge_loss_sum(u, rel, w_rel, b_rel, block=2000):
    e = u.shape[0]
    grid = e // block
    out = pl.pallas_call(
        _kge_body,
        grid=(grid,),
        in_specs=[
            pl.BlockSpec((block, D), lambda i: (i, 0)),
            pl.BlockSpec((block, D), lambda i: (i, 0)),
            pl.BlockSpec((D, D), lambda i: (0, 0)),
            pl.BlockSpec((1, D), lambda i: (0, 0)),
        ],
        out_specs=pl.BlockSpec((1, 1), lambda i: (0, 0)),
        out_shape=jax.ShapeDtypeStruct((1, 1), jnp.float32),
    )(u, rel, w_rel, b_rel.reshape(1, D))
    return out[0, 0]


def _step_body(nb_ref, g_ref, b_ref,
               wih0_ref, whh0_ref, bhh0_ref,
               wih1_ref, whh1_ref, bhh1_ref,
               ws_ref, bs_ref, h1_ref, h2_ref,
               h1o_ref, h2o_ref, sco_ref):
    g = g_ref[...]
    b = b_ref[...]
    nb = jnp.sum(nb_ref[...], axis=0)
    x = _ln(nb, g, b)

    def gru_layer(x_in, h_prev, wih, whh, bhh):
        xg = jax.lax.dot_general(x_in, wih, (((1,), (1,)), ((), ())),
                                 preferred_element_type=jnp.float32)
        hg = jax.lax.dot_general(h_prev, whh, (((1,), (1,)), ((), ())),
                                 preferred_element_type=jnp.float32) + bhh
        upd = jax.nn.sigmoid(xg[:, 0:D] + hg[:, 0:D])
        rst = jax.nn.sigmoid(xg[:, D:2 * D] + hg[:, D:2 * D])
        mem = jnp.tanh(xg[:, 2 * D:3 * D] + rst * hg[:, 2 * D:3 * D])
        return _ln((1.0 - upd) * mem + upd * h_prev, g, b)

    h1 = gru_layer(x, h1_ref[...], wih0_ref[...], whh0_ref[...], bhh0_ref[...])
    h2 = gru_layer(h1, h2_ref[...], wih1_ref[...], whh1_ref[...], bhh1_ref[...])
    h1o_ref[...] = h1
    h2o_ref[...] = h2
    sc = jnp.sum(h2 * ws_ref[...], axis=1)[None, :] + bs_ref[...]
    sco_ref[...] = sc[None, :, :]


def _step_dense(nb_parts, ln_g, ln_b, wih0, whh0, bhh0, wih1, whh1, bhh1,
                w_score, b_score, h1_prev, h2_prev, block=1000):
    p, n, _ = nb_parts.shape
    grid = n // block
    h1o, h2o, sco = pl.pallas_call(
        _step_body,
        grid=(grid,),
        in_specs=[
            pl.BlockSpec((p, block, D), lambda i: (0, i, 0)),
            pl.BlockSpec((1, D), lambda i: (0, 0)),
            pl.BlockSpec((1, D), lambda i: (0, 0)),
            pl.BlockSpec((3 * D, D), lambda i: (0, 0)),
            pl.BlockSpec((3 * D, D), lambda i: (0, 0)),
            pl.BlockSpec((1, 3 * D), lambda i: (0, 0)),
            pl.BlockSpec((3 * D, D), lambda i: (0, 0)),
            pl.BlockSpec((3 * D, D), lambda i: (0, 0)),
            pl.BlockSpec((1, 3 * D), lambda i: (0, 0)),
            pl.BlockSpec((1, D), lambda i: (0, 0)),
            pl.BlockSpec((1, 1), lambda i: (0, 0)),
            pl.BlockSpec((block, D), lambda i: (i, 0)),
            pl.BlockSpec((block, D), lambda i: (i, 0)),
        ],
        out_specs=[
            pl.BlockSpec((block, D), lambda i: (i, 0)),
            pl.BlockSpec((block, D), lambda i: (i, 0)),
            pl.BlockSpec((1, 1, block), lambda i: (i, 0, 0)),
        ],
        out_shape=[
            jax.ShapeDtypeStruct((n, D), jnp.float32),
            jax.ShapeDtypeStruct((n, D), jnp.float32),
            jax.ShapeDtypeStruct((grid, 1, block), jnp.float32),
        ],
    )(nb_parts, ln_g.reshape(1, D), ln_b.reshape(1, D),
      wih0, whh0, bhh0.reshape(1, 3 * D), wih1, whh1, bhh1.reshape(1, 3 * D),
      w_score, b_score.reshape(1, 1), h1_prev, h2_prev)
    return h1o, h2o, sco.reshape(n)


def _mask_softmax_body(sc_ref, mk_ref, lbl_ref, em_ref, out_ref):
    mk = jnp.sum(mk_ref[...], axis=0)
    im = ((mk + lbl_ref[...]) > 1e-8).astype(jnp.float32) * em_ref[...]
    s = im * sc_ref[...] + (1.0 - im) * (-1e20)
    m = jnp.max(s, axis=1, keepdims=True)
    ex = jnp.exp(s - m)
    out_ref[...] = ex / jnp.sum(ex, axis=1, keepdims=True)


def _mask_softmax(score_bm, mask_parts, prev_label, entity_mask):
    bq, mq = score_bm.shape
    p = mask_parts.shape[0]
    return pl.pallas_call(
        _mask_softmax_body,
        in_specs=[
            pl.BlockSpec((bq, mq), lambda: (0, 0)),
            pl.BlockSpec((p, bq, mq), lambda: (0, 0, 0)),
            pl.BlockSpec((bq, mq), lambda: (0, 0)),
            pl.BlockSpec((bq, mq), lambda: (0, 0)),
        ],
        out_specs=pl.BlockSpec((bq, mq), lambda: (0, 0)),
        out_shape=jax.ShapeDtypeStruct((bq, mq), jnp.float32),
    )(score_bm, mask_parts, prev_label, entity_mask)


def _ffn_body(h_ref, w_ref, b_ref, o_ref):
    o_ref[...] = jax.lax.dot_general(
        h_ref[...], w_ref[...], (((1,), (1,)), ((), ())),
        preferred_element_type=jnp.float32) + b_ref[...]


def _ffn(h, w, b, block=1000):
    n = h.shape[0]
    grid = n // block
    return pl.pallas_call(
        _ffn_body,
        grid=(grid,),
        in_specs=[
            pl.BlockSpec((block, D), lambda i: (i, 0)),
            pl.BlockSpec((D, D), lambda i: (0, 0)),
            pl.BlockSpec((1, D), lambda i: (0, 0)),
        ],
        out_specs=pl.BlockSpec((block, D), lambda i: (i, 0)),
        out_shape=jax.ShapeDtypeStruct((n, D), jnp.float32),
    )(h, w, b.reshape(1, D))


def kernel(instructions, entity_emb, fact_relations, topic_label, entity_mask,
           batch_ids, head2edge, tail2edge, ln_g, ln_b, W_rel, b_rel,
           W_ih_0, W_hh_0, b_hh_0, W_ih_1, W_hh_1, b_hh_1,
           W_score, b_score, W_ffn, b_ffn):
    S = instructions.shape[0]
    Bq, Mq, Dq = entity_emb.shape
    BM = Bq * Mq
    E = fact_relations.shape[0]

    ef = _ln_rows(entity_emb.reshape(BM, Dq), ln_g, ln_b)

    u = ef[head2edge] * ef[tail2edge]
    kge_loss = 2.0 * _kge_loss_sum(u, fact_relations, W_rel, b_rel) / E

    h1 = ef
    h2 = ef
    ent_label = topic_label
    labels = []
    for i in range(S):
        q = instructions[i]
        prior = ent_label.reshape(-1)[head2edge]
        fact_x = jax.nn.relu(q[batch_ids] * fact_relations)
        neighbor = jax.ops.segment_sum(prior[:, None] * fact_x, tail2edge,
                                       num_segments=BM)
        inter_mask = jax.ops.segment_sum(prior, tail2edge, num_segments=BM)
        nb_parts = neighbor[None]
        mask_parts = inter_mask[None].reshape(1, Bq, Mq)
        h1, h2, score = _step_dense(
            nb_parts, ln_g, ln_b, W_ih_0, W_hh_0, b_hh_0,
            W_ih_1, W_hh_1, b_hh_1, W_score, b_score, h1, h2)
        ent_label = _mask_softmax(score.reshape(Bq, Mq), mask_parts,
                                  ent_label, entity_mask)
        labels.append(ent_label)

    final = _ffn(h2, W_ffn, b_ffn).reshape(Bq, Mq, Dq)
    return (jnp.stack(labels, axis=0), final, jnp.stack([kge_loss]))
